# Initial kernel scaffold; baseline (speedup 1.0000x reference)
import jax
import jax.numpy as jnp
from jax.experimental import pallas as pl


def kernel(times, values, t):
    raise NotImplementedError("write your pallas kernel here")


# speedup vs baseline: 3.1801x; 3.1801x over previous
import jax
import jax.numpy as jnp
from jax import lax
from jax.experimental import pallas as pl
from jax.experimental.pallas import tpu as pltpu
from jax.experimental.pallas import tpu_sc as plsc

NTIME = 2048
NBATCH = 4096
LANES = 16
NWORKERS = 32
COLS_PER_WORKER = NBATCH // NWORKERS
NGROUPS = COLS_PER_WORKER // LANES


def _interp_body(times_hbm, values_hbm, t_hbm, out_hbm,
                 t_v, idx_v, idx2_v, g_v, tk_v, tk1_v, vk_v, vk1_v, out_v,
                 sem):
    nc = 2
    wid = lax.axis_index("s") * nc + lax.axis_index("c")
    base = wid * COLS_PER_WORKER

    pltpu.sync_copy(t_hbm.at[pl.ds(base, COLS_PER_WORKER)], t_v)

    lane = lax.iota(jnp.int32, LANES)
    cols = [base + (i * LANES) + lane for i in range(NGROUPS)]
    t_regs = [t_v[pl.ds(i * LANES, LANES)] for i in range(NGROUPS)]

    pos = [jnp.zeros((LANES,), jnp.int32) for _ in range(NGROUPS)]
    s = NTIME
    while s >= 1:
        for i in range(NGROUPS):
            row = jnp.minimum(pos[i] + (s - 1), NTIME - 1)
            idx_v[pl.ds(i * LANES, LANES)] = row * NBATCH + cols[i]
        pltpu.async_copy(times_hbm.at[idx_v], g_v, sem).wait()
        for i in range(NGROUPS):
            g = g_v[pl.ds(i * LANES, LANES)]
            ok = jnp.logical_and(pos[i] + s <= NTIME,
                                 g <= t_regs[i])
            pos[i] = pos[i] + jnp.where(ok, s, 0)
        s //= 2

    sels = []
    for i in range(NGROUPS):
        g0 = jnp.bitwise_and(pos[i], NTIME - 1)
        sel = g0 == 0
        sels.append(sel)
        k = jnp.where(sel, NTIME - 2, g0 - 1)
        flat = k * NBATCH + cols[i]
        idx_v[pl.ds(i * LANES, LANES)] = flat
        idx2_v[pl.ds(i * LANES, LANES)] = flat + NBATCH

    c0 = pltpu.async_copy(times_hbm.at[idx_v], tk_v, sem)
    c1 = pltpu.async_copy(times_hbm.at[idx2_v], tk1_v, sem)
    c2 = pltpu.async_copy(values_hbm.at[idx_v], vk_v, sem)
    c3 = pltpu.async_copy(values_hbm.at[idx2_v], vk1_v, sem)
    c0.wait()
    c1.wait()
    c2.wait()
    c3.wait()

    for i in range(NGROUPS):
        sl = pl.ds(i * LANES, LANES)
        tk = tk_v[sl]
        tk1 = tk1_v[sl]
        vk = vk_v[sl]
        vk1 = vk1_v[sl]
        s0 = (vk1 - vk) / (tk1 - tk)
        v0 = jnp.where(sels[i], vk1, vk)
        t0 = jnp.where(sels[i], tk1, tk)
        out_v[sl] = v0 + s0 * (t_regs[i] - t0)

    pltpu.sync_copy(out_v, out_hbm.at[pl.ds(base, COLS_PER_WORKER)])


def kernel(times, values, t):
    mesh = plsc.VectorSubcoreMesh(core_axis_name="c", subcore_axis_name="s")
    f = pl.kernel(
        _interp_body,
        mesh=mesh,
        out_type=jax.ShapeDtypeStruct((NBATCH,), jnp.float32),
        scratch_types=[
            pltpu.VMEM((COLS_PER_WORKER,), jnp.float32),
            pltpu.VMEM((COLS_PER_WORKER,), jnp.int32),
            pltpu.VMEM((COLS_PER_WORKER,), jnp.int32),
            pltpu.VMEM((COLS_PER_WORKER,), jnp.float32),
            pltpu.VMEM((COLS_PER_WORKER,), jnp.float32),
            pltpu.VMEM((COLS_PER_WORKER,), jnp.float32),
            pltpu.VMEM((COLS_PER_WORKER,), jnp.float32),
            pltpu.VMEM((COLS_PER_WORKER,), jnp.float32),
            pltpu.VMEM((COLS_PER_WORKER,), jnp.float32),
            pltpu.SemaphoreType.DMA,
        ],
    )
    return f(times.reshape(-1), values.reshape(-1), t)


# speedup vs baseline: 7.1400x; 2.2452x over previous
import jax
import jax.numpy as jnp
from jax import lax
from jax.experimental import pallas as pl
from jax.experimental.pallas import tpu as pltpu
from jax.experimental.pallas import tpu_sc as plsc

NTIME = 2048
NBATCH = 4096
LANES = 16
NWORKERS = 32
W = NBATCH // NWORKERS
NG = W // LANES
CSTRIDE = 16
NC_ROWS = NTIME // CSTRIDE


def _interp_body(times_hbm, values_hbm, t_hbm, out_hbm,
                 t_v, idx_v, idx2_v, coarse_v, fine_v, finv_v, finv1_v,
                 out_v, sem):
    nc = 2
    wid = lax.axis_index("s") * nc + lax.axis_index("c")
    base = wid * W

    lane = lax.iota(jnp.int32, LANES)

    for g in range(NG):
        idx_v[pl.ds(g * LANES, LANES)] = (lane + g * LANES) * CSTRIDE + (
            CSTRIDE - 1)
    ct = pltpu.async_copy(times_hbm.at[idx_v, pl.ds(base, W)], coarse_v, sem)
    pltpu.sync_copy(t_hbm.at[pl.ds(base, W)], t_v)
    t_regs = [t_v[pl.ds(g * LANES, LANES)] for g in range(NG)]
    ct.wait()

    loc = [lane + g * LANES for g in range(NG)]
    posc = [jnp.zeros((LANES,), jnp.int32) for _ in range(NG)]
    s = NC_ROWS
    while s >= 1:
        for g in range(NG):
            row = jnp.minimum(posc[g] + (s - 1), NC_ROWS - 1)
            val = plsc.load_gather(coarse_v, [row, loc[g]])
            ok = jnp.logical_and(posc[g] + s <= NC_ROWS, val <= t_regs[g])
            posc[g] = posc[g] + jnp.where(ok, s, 0)
        s //= 2

    pos = [p * CSTRIDE for p in posc]
    s = CSTRIDE // 2
    while s >= 1:
        for g in range(NG):
            row = jnp.minimum(pos[g] + (s - 1), NTIME - 1)
            idx_v[pl.ds(g * LANES, LANES)] = row
        pltpu.async_copy(times_hbm.at[idx_v, pl.ds(base, W)], fine_v,
                         sem).wait()
        for g in range(NG):
            val = plsc.load_gather(fine_v, [loc[g], loc[g]])
            ok = jnp.logical_and(pos[g] + s <= NTIME, val <= t_regs[g])
            pos[g] = pos[g] + jnp.where(ok, s, 0)
        s //= 2

    sels = []
    for g in range(NG):
        g0 = jnp.bitwise_and(pos[g], NTIME - 1)
        sel = g0 == 0
        sels.append(sel)
        k = jnp.where(sel, NTIME - 2, g0 - 1)
        idx_v[pl.ds(g * LANES, LANES)] = k
        idx2_v[pl.ds(g * LANES, LANES)] = k + 1
    cs = pl.ds(base, W)
    c0 = pltpu.async_copy(times_hbm.at[idx_v, cs], fine_v, sem)
    c1 = pltpu.async_copy(times_hbm.at[idx2_v, cs], coarse_v, sem)
    c2 = pltpu.async_copy(values_hbm.at[idx_v, cs], finv_v, sem)
    c3 = pltpu.async_copy(values_hbm.at[idx2_v, cs], finv1_v, sem)
    c0.wait()
    c1.wait()
    c2.wait()
    c3.wait()

    for g in range(NG):
        tk = plsc.load_gather(fine_v, [loc[g], loc[g]])
        tk1 = plsc.load_gather(coarse_v, [loc[g], loc[g]])
        vk = plsc.load_gather(finv_v, [loc[g], loc[g]])
        vk1 = plsc.load_gather(finv1_v, [loc[g], loc[g]])
        s0 = (vk1 - vk) / (tk1 - tk)
        v0 = jnp.where(sels[g], vk1, vk)
        t0 = jnp.where(sels[g], tk1, tk)
        out_v[pl.ds(g * LANES, LANES)] = v0 + s0 * (t_regs[g] - t0)

    pltpu.sync_copy(out_v, out_hbm.at[pl.ds(base, W)])


def kernel(times, values, t):
    mesh = plsc.VectorSubcoreMesh(core_axis_name="c", subcore_axis_name="s")
    f = pl.kernel(
        _interp_body,
        mesh=mesh,
        out_type=jax.ShapeDtypeStruct((NBATCH,), jnp.float32),
        compiler_params=pltpu.CompilerParams(needs_layout_passes=False),
        scratch_types=[
            pltpu.VMEM((W,), jnp.float32),
            pltpu.VMEM((W,), jnp.int32),
            pltpu.VMEM((W,), jnp.int32),
            pltpu.VMEM((NC_ROWS, W), jnp.float32),
            pltpu.VMEM((W, W), jnp.float32),
            pltpu.VMEM((W, W), jnp.float32),
            pltpu.VMEM((W, W), jnp.float32),
            pltpu.VMEM((W,), jnp.float32),
            pltpu.SemaphoreType.DMA,
        ],
    )
    return f(times, values, t)
